# Initial kernel scaffold; baseline (speedup 1.0000x reference)
import jax
import jax.numpy as jnp
from jax.experimental import pallas as pl


def kernel(x, edge_index, edge_label_index, W1, b1, W2, b2, W3, b3):
    raise NotImplementedError("write your pallas kernel here")


# speedup vs baseline: 4.9076x; 4.9076x over previous
import functools

import jax
import jax.numpy as jnp
from jax import lax
from jax.experimental import pallas as pl
from jax.experimental.pallas import tpu as pltpu
from jax.experimental.pallas import tpu_sc as plsc

N_CORES = 2
N_SUBCORES = 16
N_TILES = N_CORES * N_SUBCORES
EDGE_CHUNK = 128

_MESH = plsc.VectorSubcoreMesh(
    core_axis_name="c", subcore_axis_name="s",
    num_cores=N_CORES, num_subcores=N_SUBCORES)


@functools.partial(jax.jit, static_argnames=("n_nodes", "n_edges", "d"))
def _sc_aggregate(h, src, dst, init_rows, *, n_nodes, n_edges, d):
    n_chunks = n_edges // EDGE_CHUNK
    rpt = (n_nodes // N_SUBCORES) // 8 * 8
    rem = n_nodes - rpt * N_SUBCORES

    @functools.partial(
        pl.kernel,
        out_type=jax.ShapeDtypeStruct((N_CORES, n_nodes, d), jnp.float32),
        mesh=_MESH,
        scratch_types=[
            pltpu.VMEM((EDGE_CHUNK,), jnp.int32),
            pltpu.VMEM((EDGE_CHUNK,), jnp.int32),
            pltpu.VMEM((EDGE_CHUNK, d), jnp.float32),
            pltpu.VMEM_SHARED((n_nodes, d), jnp.float32),
        ],
    )
    def agg_kernel(h_hbm, src_hbm, dst_hbm, init_hbm, out_hbm,
                   src_v, dst_v, rows_v, acc_sh):
        cid = lax.axis_index("c")
        sid = lax.axis_index("s")
        wid = sid * N_CORES + cid

        my_rows = pl.ds(sid * rpt, rpt)
        tail_rows = pl.ds(rpt * N_SUBCORES, rem)
        pltpu.sync_copy(init_hbm.at[cid, pl.ds(0, rpt)], acc_sh.at[my_rows])

        @pl.when(sid == N_SUBCORES - 1)
        def _init_tail():
            pltpu.sync_copy(init_hbm.at[cid, pl.ds(rpt, rem)],
                            acc_sh.at[tail_rows])

        plsc.subcore_barrier()

        @pl.loop(wid, n_chunks, step=N_TILES)
        def _edge_chunk(c):
            base = c * EDGE_CHUNK
            pltpu.sync_copy(src_hbm.at[pl.ds(base, EDGE_CHUNK)], src_v)
            pltpu.sync_copy(dst_hbm.at[pl.ds(base, EDGE_CHUNK)], dst_v)
            pltpu.sync_copy(h_hbm.at[src_v], rows_v)
            pltpu.sync_copy(rows_v, acc_sh.at[dst_v], add=True)

        plsc.subcore_barrier()
        pltpu.sync_copy(acc_sh.at[my_rows], out_hbm.at[cid, my_rows])

        @pl.when(sid == N_SUBCORES - 1)
        def _out_tail():
            pltpu.sync_copy(acc_sh.at[tail_rows], out_hbm.at[cid, tail_rows])

    return agg_kernel(h, src, dst, init_rows)


@functools.partial(jax.jit, static_argnames=("n_lbl", "d"))
def _sc_decode_gather(z, src_l, dst_l, *, n_lbl, d):
    n_chunks = n_lbl // EDGE_CHUNK

    @functools.partial(
        pl.kernel,
        out_type=(jax.ShapeDtypeStruct((n_lbl, d), jnp.float32),
                  jax.ShapeDtypeStruct((n_lbl, d), jnp.float32)),
        mesh=_MESH,
        scratch_types=[
            pltpu.VMEM((EDGE_CHUNK,), jnp.int32),
            pltpu.VMEM((EDGE_CHUNK, d), jnp.float32),
        ],
    )
    def dec_kernel(z_hbm, src_hbm, dst_hbm, gs_hbm, gd_hbm, idx_v, rows_v):
        cid = lax.axis_index("c")
        sid = lax.axis_index("s")
        wid = sid * N_CORES + cid

        @pl.loop(wid, n_chunks, step=N_TILES)
        def _chunk(c):
            sl = pl.ds(c * EDGE_CHUNK, EDGE_CHUNK)
            pltpu.sync_copy(src_hbm.at[sl], idx_v)
            pltpu.sync_copy(z_hbm.at[idx_v], rows_v)
            pltpu.sync_copy(rows_v, gs_hbm.at[sl])
            pltpu.sync_copy(dst_hbm.at[sl], idx_v)
            pltpu.sync_copy(z_hbm.at[idx_v], rows_v)
            pltpu.sync_copy(rows_v, gd_hbm.at[sl])

    return dec_kernel(z, src_l, dst_l)


def _mm_first(x, w):
    def body(x_ref, w_ref, o_ref):
        o_ref[...] = jnp.dot(x_ref[...], w_ref[...],
                             preferred_element_type=jnp.float32)

    return pl.pallas_call(
        body,
        out_shape=jax.ShapeDtypeStruct((x.shape[0], w.shape[1]), jnp.float32),
    )(x, w)


def _mm_fused(parts, w, relu):
    def body(p_ref, w_ref, o_ref):
        t = p_ref[0] + p_ref[1]
        if relu:
            t = jnp.maximum(t, 0.0)
        o_ref[...] = jnp.dot(t, w_ref[...], preferred_element_type=jnp.float32)

    return pl.pallas_call(
        body,
        out_shape=jax.ShapeDtypeStruct((parts.shape[1], w.shape[1]),
                                       jnp.float32),
    )(parts, w)


def _finalize(parts):
    def body(p_ref, o_ref):
        o_ref[...] = p_ref[0] + p_ref[1]

    return pl.pallas_call(
        body,
        out_shape=jax.ShapeDtypeStruct(parts.shape[1:], jnp.float32),
    )(parts)


def _dot_rows(gs, gd):
    n, d = gs.shape
    blk = n // 16

    def body(s_ref, d_ref, o_ref):
        o_ref[...] = jnp.sum(s_ref[...] * d_ref[...], axis=1, keepdims=True)

    return pl.pallas_call(
        body,
        grid=(16,),
        in_specs=[pl.BlockSpec((blk, d), lambda i: (i, 0)),
                  pl.BlockSpec((blk, d), lambda i: (i, 0))],
        out_specs=pl.BlockSpec((blk, 1), lambda i: (i, 0)),
        out_shape=jax.ShapeDtypeStruct((n, 1), jnp.float32),
    )(gs, gd)


def kernel(x, edge_index, edge_label_index, W1, b1, W2, b2, W3, b3):
    n_nodes, d = x.shape
    n_edges = edge_index.shape[1]
    n_lbl = edge_label_index.shape[1]
    init_len = (n_nodes // N_SUBCORES) // 8 * 8 + (
        n_nodes - (n_nodes // N_SUBCORES) // 8 * 8 * N_SUBCORES)

    src = edge_index[0]
    dst = edge_index[1]

    def init_rows(b):
        bias_rows = jnp.broadcast_to(b, (1, init_len, d))
        return jnp.concatenate(
            [bias_rows, jnp.zeros((N_CORES - 1, init_len, d),
                                  jnp.float32)], axis=0)

    h1 = _mm_first(x, W1)
    p1 = _sc_aggregate(h1, src, dst, init_rows(b1),
                       n_nodes=n_nodes, n_edges=n_edges, d=d)
    h2 = _mm_fused(p1, W2, relu=True)
    p2 = _sc_aggregate(h2, src, dst, init_rows(b2),
                       n_nodes=n_nodes, n_edges=n_edges, d=d)
    h3 = _mm_fused(p2, W3, relu=True)
    p3 = _sc_aggregate(h3, src, dst, init_rows(b3),
                       n_nodes=n_nodes, n_edges=n_edges, d=d)
    z = _finalize(p3)

    n_pad = ((n_lbl + EDGE_CHUNK - 1) // EDGE_CHUNK) * EDGE_CHUNK
    pad = n_pad - n_lbl
    src_l = jnp.pad(edge_label_index[0], (0, pad))
    dst_l = jnp.pad(edge_label_index[1], (0, pad))
    gs, gd = _sc_decode_gather(z, src_l, dst_l, n_lbl=n_pad, d=d)
    dots = _dot_rows(gs, gd)
    return dots[:n_lbl, 0]
